# baseline (speedup 1.0000x reference)
import functools

import jax
import jax.numpy as jnp
from jax import lax
from jax.experimental import pallas as pl
from jax.experimental.pallas import tpu as pltpu
from jax.experimental.pallas import tpu_sc as plsc

_BATCH = 1024
_LAT_DIM = 64
_VOCAB = 256
_CUR_DIM = 32

_NC = 2
_NS = 16
_L = 16
_NW = _NC * _NS
_BPW = _BATCH // _NW
_UNROLL = 8


def _bounds_body(table_hbm, low_hbm, upp_hbm, sj_hbm, out_low_hbm, out_upp_hbm,
                 rows_v, sj_v, low_v, upp_v, olow_v, oupp_v, sem):
    wid = lax.axis_index("s") * _NC + lax.axis_index("c")
    base = wid * _BPW
    iota = lax.iota(jnp.int32, _L)

    copy = pltpu.async_copy(table_hbm.at[pl.ds(base, _BPW), _CUR_DIM], rows_v, sem)
    pltpu.sync_copy(sj_hbm.at[pl.ds(base, _BPW)], sj_v)
    pltpu.sync_copy(low_hbm.at[pl.ds(base, _BPW)], low_v)
    pltpu.sync_copy(upp_hbm.at[pl.ds(base, _BPW)], upp_v)
    copy.wait()

    for g in range(_BPW // _L):
        rowsel = iota + (g * _L)
        sj = sj_v[pl.ds(g * _L, _L)]

        def body(step, acc, rowsel=rowsel, sj=sj):
            v0 = step * _UNROLL
            for u in range(_UNROLL):
                v = v0 + u
                col = plsc.load_gather(rows_v, [rowsel, jnp.full((_L,), v, jnp.int32)])
                acc = acc + jnp.where(v < sj, col, jnp.float32(0.0))
            return acc

        cdf_low = lax.fori_loop(0, _VOCAB // _UNROLL, body,
                                jnp.zeros((_L,), jnp.float32))
        p_at = plsc.load_gather(rows_v, [rowsel, sj])
        low = low_v[pl.ds(g * _L, _L)]
        upp = upp_v[pl.ds(g * _L, _L)]
        rng = upp - low
        olow_v[pl.ds(g * _L, _L)] = low + rng * cdf_low
        oupp_v[pl.ds(g * _L, _L)] = low + rng * (cdf_low + p_at)

    pltpu.sync_copy(olow_v, out_low_hbm.at[pl.ds(base, _BPW)])
    pltpu.sync_copy(oupp_v, out_upp_hbm.at[pl.ds(base, _BPW)])


_sc_update_bounds = functools.partial(
    pl.kernel,
    mesh=plsc.VectorSubcoreMesh(core_axis_name="c", subcore_axis_name="s"),
    compiler_params=pltpu.CompilerParams(use_tc_tiling_on_sc=False,
                                         needs_layout_passes=False),
    out_type=(jax.ShapeDtypeStruct((_BATCH,), jnp.float32),
              jax.ShapeDtypeStruct((_BATCH,), jnp.float32)),
    scratch_types=[
        pltpu.VMEM((_BPW, _VOCAB), jnp.float32),
        pltpu.VMEM((_BPW,), jnp.int32),
        pltpu.VMEM((_BPW,), jnp.float32),
        pltpu.VMEM((_BPW,), jnp.float32),
        pltpu.VMEM((_BPW,), jnp.float32),
        pltpu.VMEM((_BPW,), jnp.float32),
        pltpu.SemaphoreType.DMA,
    ],
)(_bounds_body)


def kernel(low_bound, upp_bound, softmax, s_j):
    sj = s_j.astype(jnp.int32)
    new_low, new_upp = _sc_update_bounds(softmax, low_bound, upp_bound, sj)
    return (new_low, new_upp)


# speedup vs baseline: 2.6004x; 2.6004x over previous
import functools

import jax
import jax.numpy as jnp
from jax import lax
from jax.experimental import pallas as pl
from jax.experimental.pallas import tpu as pltpu
from jax.experimental.pallas import tpu_sc as plsc

_BATCH = 1024
_LAT_DIM = 64
_VOCAB = 256
_CUR_DIM = 32

_NC = 2
_NS = 16
_L = 16
_NW = _NC * _NS
_BPW = _BATCH // _NW
_UNROLL = 8


_TC_BB = 128


def _slice_t_body(src_ref, dst_ref):
    dst_ref[...] = src_ref[:, _CUR_DIM % 8, :].T


_extract_t = pl.pallas_call(
    _slice_t_body,
    grid=(_BATCH // _TC_BB,),
    in_specs=[pl.BlockSpec((_TC_BB, 8, _VOCAB), lambda i: (i, _CUR_DIM // 8, 0))],
    out_specs=pl.BlockSpec((_VOCAB, _TC_BB), lambda i: (0, i)),
    out_shape=jax.ShapeDtypeStruct((_VOCAB, _BATCH), jnp.float32),
)


def _bounds_body(pt_hbm, low_hbm, upp_hbm, sj_hbm, out_low_hbm, out_upp_hbm,
                 cols_v, sj_v, low_v, upp_v, olow_v, oupp_v, sem):
    wid = lax.axis_index("s") * _NC + lax.axis_index("c")
    base = wid * _BPW
    iota = lax.iota(jnp.int32, _L)

    copy = pltpu.async_copy(pt_hbm.at[:, pl.ds(base, _BPW)], cols_v, sem)
    pltpu.sync_copy(sj_hbm.at[pl.ds(base, _BPW)], sj_v)
    pltpu.sync_copy(low_hbm.at[pl.ds(base, _BPW)], low_v)
    pltpu.sync_copy(upp_hbm.at[pl.ds(base, _BPW)], upp_v)
    copy.wait()

    for g in range(_BPW // _L):
        rowsel = iota + (g * _L)
        sj = sj_v[pl.ds(g * _L, _L)]

        def body(step, acc, rowsel=rowsel, sj=sj):
            v0 = step * _UNROLL
            for u in range(_UNROLL):
                v = v0 + u
                col = plsc.load_gather(cols_v, [jnp.full((_L,), v, jnp.int32), rowsel])
                acc = acc + jnp.where(v < sj, col, jnp.float32(0.0))
            return acc

        cdf_low = lax.fori_loop(0, _VOCAB // _UNROLL, body,
                                jnp.zeros((_L,), jnp.float32))
        p_at = plsc.load_gather(cols_v, [sj, rowsel])
        low = low_v[pl.ds(g * _L, _L)]
        upp = upp_v[pl.ds(g * _L, _L)]
        rng = upp - low
        olow_v[pl.ds(g * _L, _L)] = low + rng * cdf_low
        oupp_v[pl.ds(g * _L, _L)] = low + rng * (cdf_low + p_at)

    pltpu.sync_copy(olow_v, out_low_hbm.at[pl.ds(base, _BPW)])
    pltpu.sync_copy(oupp_v, out_upp_hbm.at[pl.ds(base, _BPW)])


_sc_update_bounds = functools.partial(
    pl.kernel,
    mesh=plsc.VectorSubcoreMesh(core_axis_name="c", subcore_axis_name="s"),
    compiler_params=pltpu.CompilerParams(use_tc_tiling_on_sc=False,
                                         needs_layout_passes=False),
    out_type=(jax.ShapeDtypeStruct((_BATCH,), jnp.float32),
              jax.ShapeDtypeStruct((_BATCH,), jnp.float32)),
    scratch_types=[
        pltpu.VMEM((_VOCAB, _BPW), jnp.float32),
        pltpu.VMEM((_BPW,), jnp.int32),
        pltpu.VMEM((_BPW,), jnp.float32),
        pltpu.VMEM((_BPW,), jnp.float32),
        pltpu.VMEM((_BPW,), jnp.float32),
        pltpu.VMEM((_BPW,), jnp.float32),
        pltpu.SemaphoreType.DMA,
    ],
)(_bounds_body)


def kernel(low_bound, upp_bound, softmax, s_j):
    probs_t = _extract_t(softmax)
    sj = s_j.astype(jnp.int32)
    new_low, new_upp = _sc_update_bounds(probs_t, low_bound, upp_bound, sj)
    return (new_low, new_upp)


# speedup vs baseline: 2.8574x; 1.0988x over previous
import functools

import jax
import jax.numpy as jnp
from jax import lax
from jax.experimental import pallas as pl
from jax.experimental.pallas import tpu as pltpu
from jax.experimental.pallas import tpu_sc as plsc

_BATCH = 1024
_LAT_DIM = 64
_VOCAB = 256
_CUR_DIM = 32

_NC = 2
_NS = 16
_L = 16
_NW = _NC * _NS
_BPW = _BATCH // _NW
_UNROLL = 8


def _slice_t_body(src_ref, dst_ref):
    dst_ref[...] = src_ref[:, _CUR_DIM % 8, :].T


_extract_t = pl.pallas_call(
    _slice_t_body,
    grid=(1,),
    in_specs=[pl.BlockSpec((_BATCH, 8, _VOCAB), lambda i: (0, _CUR_DIM // 8, 0))],
    out_specs=pl.BlockSpec((_VOCAB, _BATCH), lambda i: (0, 0)),
    out_shape=jax.ShapeDtypeStruct((_VOCAB, _BATCH), jnp.float32),
)


def _bounds_body(pt_hbm, low_hbm, upp_hbm, sj_hbm, out_low_hbm, out_upp_hbm,
                 cols_v, sj_v, low_v, upp_v, olow_v, oupp_v, sem):
    wid = lax.axis_index("s") * _NC + lax.axis_index("c")
    base = wid * _BPW
    iota = lax.iota(jnp.int32, _L)

    copy = pltpu.async_copy(pt_hbm.at[:, pl.ds(base, _BPW)], cols_v, sem)
    pltpu.sync_copy(sj_hbm.at[pl.ds(base, _BPW)], sj_v)
    pltpu.sync_copy(low_hbm.at[pl.ds(base, _BPW)], low_v)
    pltpu.sync_copy(upp_hbm.at[pl.ds(base, _BPW)], upp_v)
    copy.wait()

    for g in range(_BPW // _L):
        rowsel = iota + (g * _L)
        sj = sj_v[pl.ds(g * _L, _L)]

        def body(step, acc, rowsel=rowsel, sj=sj):
            v0 = step * _UNROLL
            for u in range(_UNROLL):
                v = v0 + u
                col = plsc.load_gather(cols_v, [jnp.full((_L,), v, jnp.int32), rowsel])
                acc = acc + jnp.where(v < sj, col, jnp.float32(0.0))
            return acc

        cdf_low = lax.fori_loop(0, _VOCAB // _UNROLL, body,
                                jnp.zeros((_L,), jnp.float32))
        p_at = plsc.load_gather(cols_v, [sj, rowsel])
        low = low_v[pl.ds(g * _L, _L)]
        upp = upp_v[pl.ds(g * _L, _L)]
        rng = upp - low
        olow_v[pl.ds(g * _L, _L)] = low + rng * cdf_low
        oupp_v[pl.ds(g * _L, _L)] = low + rng * (cdf_low + p_at)

    pltpu.sync_copy(olow_v, out_low_hbm.at[pl.ds(base, _BPW)])
    pltpu.sync_copy(oupp_v, out_upp_hbm.at[pl.ds(base, _BPW)])


_sc_update_bounds = functools.partial(
    pl.kernel,
    mesh=plsc.VectorSubcoreMesh(core_axis_name="c", subcore_axis_name="s"),
    compiler_params=pltpu.CompilerParams(use_tc_tiling_on_sc=False,
                                         needs_layout_passes=False),
    out_type=(jax.ShapeDtypeStruct((_BATCH,), jnp.float32),
              jax.ShapeDtypeStruct((_BATCH,), jnp.float32)),
    scratch_types=[
        pltpu.VMEM((_VOCAB, _BPW), jnp.float32),
        pltpu.VMEM((_BPW,), jnp.int32),
        pltpu.VMEM((_BPW,), jnp.float32),
        pltpu.VMEM((_BPW,), jnp.float32),
        pltpu.VMEM((_BPW,), jnp.float32),
        pltpu.VMEM((_BPW,), jnp.float32),
        pltpu.SemaphoreType.DMA,
    ],
)(_bounds_body)


def kernel(low_bound, upp_bound, softmax, s_j):
    probs_t = _extract_t(softmax)
    sj = s_j.astype(jnp.int32)
    new_low, new_upp = _sc_update_bounds(probs_t, low_bound, upp_bound, sj)
    return (new_low, new_upp)


# speedup vs baseline: 2.8582x; 1.0003x over previous
import functools

import jax
import jax.numpy as jnp
from jax import lax
from jax.experimental import pallas as pl
from jax.experimental.pallas import tpu as pltpu
from jax.experimental.pallas import tpu_sc as plsc

_BATCH = 1024
_LAT_DIM = 64
_VOCAB = 256
_CUR_DIM = 32

_NC = 2
_NS = 16
_L = 16
_NW = _NC * _NS
_BPW = _BATCH // _NW
_UNROLL = 8


def _slice_t_body(src_ref, dst_ref):
    dst_ref[...] = src_ref[:, _CUR_DIM % 8, :].T


_extract_t = pl.pallas_call(
    _slice_t_body,
    grid=(1,),
    in_specs=[pl.BlockSpec((_BATCH, 8, _VOCAB), lambda i: (0, _CUR_DIM // 8, 0))],
    out_specs=pl.BlockSpec((_VOCAB, _BATCH), lambda i: (0, 0)),
    out_shape=jax.ShapeDtypeStruct((_VOCAB, _BATCH), jnp.float32),
)


def _bounds_body(pt_hbm, low_hbm, upp_hbm, sj_hbm, out_low_hbm, out_upp_hbm,
                 cols_v, sj_v, low_v, upp_v, olow_v, oupp_v, sem):
    wid = lax.axis_index("s") * _NC + lax.axis_index("c")
    base = wid * _BPW
    iota = lax.iota(jnp.int32, _L)

    copy = pltpu.async_copy(pt_hbm.at[:, pl.ds(base, _BPW)], cols_v, sem)
    pltpu.sync_copy(sj_hbm.at[pl.ds(base, _BPW)], sj_v)
    pltpu.sync_copy(low_hbm.at[pl.ds(base, _BPW)], low_v)
    pltpu.sync_copy(upp_hbm.at[pl.ds(base, _BPW)], upp_v)
    copy.wait()

    for g in range(_BPW // _L):
        rowsel = iota + (g * _L)
        sj = sj_v[pl.ds(g * _L, _L)]

        def body(step, acc, rowsel=rowsel, sj=sj):
            v0 = step * _UNROLL
            for u in range(_UNROLL):
                v = v0 + u
                col = plsc.load_gather(cols_v, [jnp.full((_L,), v, jnp.int32), rowsel])
                acc = acc + jnp.where(v < sj, col, jnp.float32(0.0))
            return acc

        cdf_low = lax.fori_loop(0, _VOCAB // _UNROLL, body,
                                jnp.zeros((_L,), jnp.float32))
        p_at = plsc.load_gather(cols_v, [sj, rowsel])
        low = low_v[pl.ds(g * _L, _L)]
        upp = upp_v[pl.ds(g * _L, _L)]
        rng = upp - low
        olow_v[pl.ds(g * _L, _L)] = low + rng * cdf_low
        oupp_v[pl.ds(g * _L, _L)] = low + rng * (cdf_low + p_at)

    pltpu.sync_copy(olow_v, out_low_hbm.at[pl.ds(base, _BPW)])
    pltpu.sync_copy(oupp_v, out_upp_hbm.at[pl.ds(base, _BPW)])


_sc_update_bounds = functools.partial(
    pl.kernel,
    mesh=plsc.VectorSubcoreMesh(core_axis_name="c", subcore_axis_name="s"),
    compiler_params=pltpu.CompilerParams(use_tc_tiling_on_sc=False,
                                         needs_layout_passes=False,
                                         skip_device_barrier=True),
    out_type=(jax.ShapeDtypeStruct((_BATCH,), jnp.float32),
              jax.ShapeDtypeStruct((_BATCH,), jnp.float32)),
    scratch_types=[
        pltpu.VMEM((_VOCAB, _BPW), jnp.float32),
        pltpu.VMEM((_BPW,), jnp.int32),
        pltpu.VMEM((_BPW,), jnp.float32),
        pltpu.VMEM((_BPW,), jnp.float32),
        pltpu.VMEM((_BPW,), jnp.float32),
        pltpu.VMEM((_BPW,), jnp.float32),
        pltpu.SemaphoreType.DMA,
    ],
)(_bounds_body)


def kernel(low_bound, upp_bound, softmax, s_j):
    probs_t = _extract_t(softmax)
    sj = s_j.astype(jnp.int32)
    new_low, new_upp = _sc_update_bounds(probs_t, low_bound, upp_bound, sj)
    return (new_low, new_upp)
